# baseline (speedup 1.0000x reference)
import functools

import jax
import jax.numpy as jnp
from jax import lax
from jax.experimental import pallas as pl
from jax.experimental.pallas import tpu as pltpu
from jax.experimental.pallas import tpu_sc as plsc

N_NODES = 1000000
MEM_DIM = 64
BATCH = 16384
TIME_INIT = -1.0

_NUM_CORES = 2
_NUM_SUBCORES = 16
_NW = _NUM_CORES * _NUM_SUBCORES
_BPW = BATCH // _NW
_LANES = 16
_VPR = MEM_DIM // _LANES
_CH = 64
_NCHUNK = _BPW // _CH

_mesh = plsc.VectorSubcoreMesh(core_axis_name="c", subcore_axis_name="s")


@functools.partial(
    pl.kernel,
    mesh=_mesh,
    out_type=jax.ShapeDtypeStruct((BATCH, MEM_DIM), jnp.float32),
    scratch_types=[
        pltpu.VMEM((_BPW,), jnp.int32),
        pltpu.VMEM((_BPW,), jnp.float32),
        pltpu.VMEM((MEM_DIM,), jnp.float32),
        pltpu.VMEM((_BPW, MEM_DIM), jnp.float32),
        pltpu.SemaphoreType.DMA,
        pltpu.SemaphoreType.DMA,
    ],
)
def _gather_mem(mem_hbm, lu_hbm, dflt_hbm, idx_hbm, out_hbm,
                idx_v, lu_v, dflt_v, rows_v, sem_g, sem_lu):
    wid = lax.axis_index("s") * _NUM_CORES + lax.axis_index("c")
    base = wid * _BPW

    pltpu.sync_copy(idx_hbm.at[pl.ds(base, _BPW)], idx_v)
    pltpu.sync_copy(dflt_hbm, dflt_v)
    cp_lu = pltpu.async_copy(lu_hbm.at[idx_v], lu_v, sem_lu)

    def batch_body(c, carry):
        idxvecs = [idx_v[pl.ds(c * _CH + g * _LANES, _LANES)]
                   for g in range(_CH // _LANES)]
        cps = []
        for i in range(_CH):
            nid = idxvecs[i // _LANES][i % _LANES]
            cps.append(pltpu.async_copy(
                mem_hbm.at[nid], rows_v.at[c * _CH + i], sem_g))
        for cp in cps:
            cp.wait()
        return carry

    lax.fori_loop(0, _NCHUNK, batch_body, 0)
    cp_lu.wait()

    dvecs = [dflt_v[pl.ds(j * _LANES, _LANES)] for j in range(_VPR)]

    def chunk_fix(c, carry):
        lu16 = lu_v[pl.ds(c * _LANES, _LANES)]
        for i in range(_LANES):
            @pl.when(lu16[i] == jnp.float32(TIME_INIT))
            def _():
                for j in range(_VPR):
                    rows_v[c * _LANES + i, pl.ds(j * _LANES, _LANES)] = dvecs[j]
        return carry

    lax.fori_loop(0, _BPW // _LANES, chunk_fix, 0)

    pltpu.sync_copy(rows_v, out_hbm.at[pl.ds(base, _BPW)])


def kernel(memory, last_update, default_memory, node_ids):
    idx = node_ids.astype(jnp.int32)
    return _gather_mem(memory, last_update, default_memory, idx)


# speedup vs baseline: 2.2233x; 2.2233x over previous
import functools

import jax
import jax.numpy as jnp
from jax import lax
from jax.experimental import pallas as pl
from jax.experimental.pallas import tpu as pltpu
from jax.experimental.pallas import tpu_sc as plsc

N_NODES = 1000000
MEM_DIM = 64
BATCH = 16384
TIME_INIT = -1.0

_NUM_CORES = 2
_NUM_SUBCORES = 16
_NW = _NUM_CORES * _NUM_SUBCORES
_BPW = BATCH // _NW
_LANES = 16
_VPR = MEM_DIM // _LANES
_NGRP = (N_NODES + 127) // 128
_MAXOWN = (_NGRP + _NW - 1) // _NW + 1
_W = 4
_MAXWAVES = (_MAXOWN + _W - 1) // _W
_NVREG = BATCH // _LANES
_PAD = 16

_mesh = plsc.VectorSubcoreMesh(core_axis_name="c", subcore_axis_name="s")


def _sc1(v):
    return v[0]


@functools.partial(
    pl.kernel,
    mesh=_mesh,
    compiler_params=pltpu.CompilerParams(needs_layout_passes=False),
    out_type=jax.ShapeDtypeStruct((BATCH, MEM_DIM), jnp.float32),
    scratch_types=[
        pltpu.VMEM((BATCH,), jnp.int32),
        pltpu.VMEM((BATCH + _PAD,), jnp.int32),
        pltpu.VMEM((BATCH + _PAD,), jnp.int32),
        pltpu.VMEM((256 + _PAD,), jnp.int32),
        pltpu.VMEM((256 + _PAD,), jnp.int32),
        pltpu.VMEM((256 + _PAD,), jnp.int32),
        pltpu.VMEM((256 + _PAD,), jnp.int32),
        pltpu.VMEM((256 + _PAD,), jnp.int32),
        pltpu.VMEM((2, _W, MEM_DIM, 128), jnp.float32),
        pltpu.VMEM((_LANES, MEM_DIM), jnp.float32),
        pltpu.VMEM((MEM_DIM,), jnp.int32),
        pltpu.SemaphoreType.DMA,
        pltpu.SemaphoreType.DMA,
        pltpu.SemaphoreType.DMA,
    ],
)
def _route_gather(memT_hbm, idx_hbm, mid_hbm,
                  idx_all, poslist, sorted_pos, counts, starts, startsmut,
                  slots, glist, gbuf, rows16, dummy_v, sem_g0, sem_g1,
                  sem_out):
    wid = lax.axis_index("s") * _NUM_CORES + lax.axis_index("c")
    lo = (_NGRP * wid) >> 5
    hi = (_NGRP * (wid + 1)) >> 5

    pltpu.sync_copy(idx_hbm, idx_all)

    iota16 = lax.iota(jnp.int32, _LANES)
    zeros16 = jnp.zeros((_LANES,), jnp.int32)
    ones16 = jnp.full((_LANES,), 1, jnp.int32)
    lane0 = iota16 == 0

    for t in range((256 + _PAD) // _LANES):
        counts[pl.ds(t * _LANES, _LANES)] = zeros16

    def scan_body(c, cnt):
        v16 = idx_all[pl.ds(c * _LANES, _LANES)]
        nc16 = lax.shift_right_logical(v16, 7)
        m = jnp.logical_and(nc16 >= lo, nc16 < hi)
        lsafe = jnp.where(m, nc16 - lo, 0)
        plsc.addupdate_scatter(counts, [lsafe], ones16, mask=m)
        plsc.store_compressed(poslist.at[pl.ds(cnt, _LANES)],
                              c * _LANES + iota16, mask=m)
        return cnt + _sc1(plsc.all_reduce_population_count(m))

    cnt = lax.fori_loop(0, _NVREG, scan_body, 0)

    def prefix_body(t, carry):
        id_carry, g_carry = carry
        cv = counts[pl.ds(t * _LANES, _LANES)]
        inc = plsc.cumsum(cv)
        starts[pl.ds(t * _LANES, _LANES)] = inc - cv + id_carry
        startsmut[pl.ds(t * _LANES, _LANES)] = inc - cv + id_carry
        present = (cv > 0).astype(jnp.int32)
        pinc = plsc.cumsum(present)
        slots[pl.ds(t * _LANES, _LANES)] = pinc - present + g_carry
        plsc.store_compressed(glist.at[pl.ds(g_carry, _LANES)],
                              t * _LANES + iota16, mask=cv > 0)
        return (id_carry + inc[_LANES - 1], g_carry + pinc[_LANES - 1])

    _, ngroups = lax.fori_loop(0, 256 // _LANES, prefix_body, (0, 0))

    def place_body(q, carry):
        n_here = jnp.minimum(cnt - q * _LANES, _LANES)
        pos16 = poslist[pl.ds(q * _LANES, _LANES)]
        nid16 = plsc.load_gather(
            idx_all, [lax.bitwise_and(pos16, BATCH - 1)])
        local16 = jnp.minimum(jnp.maximum(
            lax.shift_right_logical(nid16, 7) - lo, 0), 255)
        for i in range(_LANES):
            @pl.when(i < n_here)
            def _():
                l = local16[i]
                d = lax.bitwise_and(_sc1(startsmut[pl.ds(l, _LANES)]),
                                    BATCH - 1)
                plsc.store_scatter(sorted_pos, [jnp.full((_LANES,), d)],
                                   jnp.full((_LANES,), pos16[i]), mask=lane0)
                plsc.store_scatter(startsmut, [jnp.full((_LANES,), l)],
                                   jnp.full((_LANES,), d + 1), mask=lane0)
        return carry

    lax.fori_loop(0, (cnt + _LANES - 1) // _LANES, place_body, 0)

    def seg_start(k):
        graw = _sc1(glist[pl.ds(jnp.minimum(k, 255), _LANES)])
        g = jnp.where(k < ngroups,
                      jnp.minimum(jnp.maximum(graw, 0), 255), 0)
        s = _sc1(starts[pl.ds(g, _LANES)])
        return jnp.where(k < ngroups, s, cnt)

    def group_copy(k, b, phase):
        graw = _sc1(glist[pl.ds(jnp.minimum(k, 255), _LANES)])
        g = jnp.minimum(jnp.maximum(graw, 0), 255)
        off = pl.multiple_of((lo + g) * 128, 128)
        sem = sem_g0 if phase == 0 else sem_g1
        return pltpu.make_async_copy(
            memT_hbm.at[:, pl.ds(off, 128)], gbuf.at[phase, b], sem)

    def fire_wave(w, phase):
        for b in range(_W):
            @pl.when(w * _W + b < ngroups)
            def _():
                group_copy(w * _W + b, b, phase).start()

    def drain_wave(w, phase):
        for b in range(_W):
            @pl.when(w * _W + b < ngroups)
            def _():
                group_copy(w * _W + b, b, phase).wait()

    dummy_cp = pltpu.make_async_copy(idx_hbm.at[pl.ds(0, MEM_DIM)],
                                     dummy_v, sem_out)

    fire_wave(0, 0)

    def extract_wave(w, phase):
        r0 = seg_start(w * _W)
        r1 = seg_start((w + 1) * _W)

        def chunk_body(q, ccarry):
            r = r0 + q * _LANES
            n_here = jnp.minimum(r1 - r, _LANES)
            mlane = iota16 < n_here
            pos16 = sorted_pos[pl.ds(r, _LANES)]
            possafe = lax.bitwise_and(pos16, BATCH - 1)
            nid16 = plsc.load_gather(idx_all, [possafe])
            local16 = jnp.minimum(jnp.maximum(jnp.where(
                mlane, lax.shift_right_logical(nid16, 7) - lo, 0), 0), 255)
            slot16 = plsc.load_gather(slots, [local16])
            b16 = jnp.minimum(jnp.maximum(
                jnp.where(mlane, slot16 - w * _W, 0), 0), _W - 1)
            dn16 = jnp.where(mlane, lax.bitwise_and(nid16, 127), 0)
            ph16 = jnp.full((_LANES,), phase)
            for f in range(MEM_DIM):
                vals = plsc.load_gather(
                    gbuf, [ph16, b16, jnp.full((_LANES,), f), dn16])
                plsc.store_scatter(rows16, [iota16, jnp.full((_LANES,), f)],
                                   vals, mask=mlane)
            for i in range(_LANES):
                @pl.when(i < n_here)
                def _():
                    pltpu.async_copy(rows16.at[i], mid_hbm.at[possafe[i]],
                                     sem_out)

            def drain_body(i, dcarry):
                dummy_cp.wait()
                return dcarry

            lax.fori_loop(0, n_here, drain_body, 0)
            return ccarry

        lax.fori_loop(0, (r1 - r0 + _LANES - 1) // _LANES, chunk_body, 0)

    def pair_body(u, ucarry):
        for ph in range(2):
            w = 2 * u + ph
            fire_wave(w + 1, 1 - ph)
            drain_wave(w, ph)
            extract_wave(w, ph)
        return ucarry

    lax.fori_loop(0, _MAXWAVES // 2, pair_body, 0)


@functools.partial(
    pl.kernel,
    mesh=_mesh,
    out_type=jax.ShapeDtypeStruct((BATCH, MEM_DIM), jnp.float32),
    scratch_types=[
        pltpu.VMEM((_BPW,), jnp.int32),
        pltpu.VMEM((_BPW,), jnp.float32),
        pltpu.VMEM((MEM_DIM,), jnp.float32),
        pltpu.VMEM((_BPW, MEM_DIM), jnp.float32),
        pltpu.SemaphoreType.DMA,
    ],
)
def _default_fixup(mid_hbm, lu_hbm, dflt_hbm, idx_hbm, out_hbm,
                   idx_v, lu_v, dflt_v, rows_v, sem_lu):
    wid = lax.axis_index("s") * _NUM_CORES + lax.axis_index("c")
    base = wid * _BPW

    pltpu.sync_copy(idx_hbm.at[pl.ds(base, _BPW)], idx_v)
    pltpu.sync_copy(dflt_hbm, dflt_v)
    cp_lu = pltpu.async_copy(lu_hbm.at[idx_v], lu_v, sem_lu)
    pltpu.sync_copy(mid_hbm.at[pl.ds(base, _BPW)], rows_v)
    cp_lu.wait()

    dvecs = [dflt_v[pl.ds(j * _LANES, _LANES)] for j in range(_VPR)]

    def chunk_fix(c, carry):
        lu16 = lu_v[pl.ds(c * _LANES, _LANES)]
        for i in range(_LANES):
            @pl.when(lu16[i] == jnp.float32(TIME_INIT))
            def _():
                for j in range(_VPR):
                    rows_v[c * _LANES + i, pl.ds(j * _LANES, _LANES)] = dvecs[j]
        return carry

    lax.fori_loop(0, _BPW // _LANES, chunk_fix, 0)

    pltpu.sync_copy(rows_v, out_hbm.at[pl.ds(base, _BPW)])


def kernel(memory, last_update, default_memory, node_ids):
    idx = node_ids.astype(jnp.int32)
    mid = _route_gather(memory.T, idx)
    return _default_fixup(mid, last_update, default_memory, idx)


# speedup vs baseline: 2.2268x; 1.0016x over previous
import functools

import jax
import jax.numpy as jnp
from jax import lax
from jax.experimental import pallas as pl
from jax.experimental.pallas import tpu as pltpu
from jax.experimental.pallas import tpu_sc as plsc

N_NODES = 1000000
MEM_DIM = 64
BATCH = 16384
TIME_INIT = -1.0

_NUM_CORES = 2
_NUM_SUBCORES = 16
_NW = _NUM_CORES * _NUM_SUBCORES
_BPW = BATCH // _NW
_LANES = 16
_VPR = MEM_DIM // _LANES
_NGRP = (N_NODES + 127) // 128
_MAXOWN = (_NGRP + _NW - 1) // _NW + 1
_W = 4
_MAXWAVES = (_MAXOWN + _W - 1) // _W
_NVREG = BATCH // _LANES
_PAD = 16

_mesh = plsc.VectorSubcoreMesh(core_axis_name="c", subcore_axis_name="s")


def _sc1(v):
    return v[0]


@functools.partial(
    pl.kernel,
    mesh=_mesh,
    compiler_params=pltpu.CompilerParams(needs_layout_passes=False),
    out_type=jax.ShapeDtypeStruct((BATCH, MEM_DIM), jnp.float32),
    scratch_types=[
        pltpu.VMEM((BATCH,), jnp.int32),
        pltpu.VMEM((BATCH + _PAD,), jnp.int32),
        pltpu.VMEM((BATCH + _PAD,), jnp.int32),
        pltpu.VMEM((256 + _PAD,), jnp.int32),
        pltpu.VMEM((256 + _PAD,), jnp.int32),
        pltpu.VMEM((256 + _PAD,), jnp.int32),
        pltpu.VMEM((256 + _PAD,), jnp.int32),
        pltpu.VMEM((256 + _PAD,), jnp.int32),
        pltpu.VMEM((2, _W, MEM_DIM, 128), jnp.float32),
        pltpu.VMEM((4, _LANES, MEM_DIM), jnp.float32),
        pltpu.VMEM((MEM_DIM,), jnp.int32),
        pltpu.SemaphoreType.DMA,
        pltpu.SemaphoreType.DMA,
        pltpu.SemaphoreType.DMA,
    ],
)
def _route_gather(memT_hbm, idx_hbm, mid_hbm,
                  idx_all, poslist, sorted_pos, counts, starts, startsmut,
                  slots, glist, gbuf, rows16, dummy_v, sem_g0, sem_g1,
                  sem_out):
    wid = lax.axis_index("s") * _NUM_CORES + lax.axis_index("c")
    lo = (_NGRP * wid) >> 5
    hi = (_NGRP * (wid + 1)) >> 5

    pltpu.sync_copy(idx_hbm, idx_all)

    iota16 = lax.iota(jnp.int32, _LANES)
    zeros16 = jnp.zeros((_LANES,), jnp.int32)
    ones16 = jnp.full((_LANES,), 1, jnp.int32)
    lane0 = iota16 == 0

    for t in range((256 + _PAD) // _LANES):
        counts[pl.ds(t * _LANES, _LANES)] = zeros16

    def scan_body(c, cnt):
        v16 = idx_all[pl.ds(c * _LANES, _LANES)]
        nc16 = lax.shift_right_logical(v16, 7)
        m = jnp.logical_and(nc16 >= lo, nc16 < hi)
        lsafe = jnp.where(m, nc16 - lo, 0)
        plsc.addupdate_scatter(counts, [lsafe], ones16, mask=m)
        plsc.store_compressed(poslist.at[pl.ds(cnt, _LANES)],
                              c * _LANES + iota16, mask=m)
        return cnt + _sc1(plsc.all_reduce_population_count(m))

    cnt = lax.fori_loop(0, _NVREG, scan_body, 0)

    def prefix_body(t, carry):
        id_carry, g_carry = carry
        cv = counts[pl.ds(t * _LANES, _LANES)]
        inc = plsc.cumsum(cv)
        starts[pl.ds(t * _LANES, _LANES)] = inc - cv + id_carry
        startsmut[pl.ds(t * _LANES, _LANES)] = inc - cv + id_carry
        present = (cv > 0).astype(jnp.int32)
        pinc = plsc.cumsum(present)
        slots[pl.ds(t * _LANES, _LANES)] = pinc - present + g_carry
        plsc.store_compressed(glist.at[pl.ds(g_carry, _LANES)],
                              t * _LANES + iota16, mask=cv > 0)
        return (id_carry + inc[_LANES - 1], g_carry + pinc[_LANES - 1])

    _, ngroups = lax.fori_loop(0, 256 // _LANES, prefix_body, (0, 0))

    def place_body(q, carry):
        n_here = jnp.minimum(cnt - q * _LANES, _LANES)
        pos16 = poslist[pl.ds(q * _LANES, _LANES)]
        nid16 = plsc.load_gather(
            idx_all, [lax.bitwise_and(pos16, BATCH - 1)])
        local16 = jnp.minimum(jnp.maximum(
            lax.shift_right_logical(nid16, 7) - lo, 0), 255)
        for i in range(_LANES):
            @pl.when(i < n_here)
            def _():
                l = local16[i]
                d = lax.bitwise_and(_sc1(startsmut[pl.ds(l, _LANES)]),
                                    BATCH - 1)
                plsc.store_scatter(sorted_pos, [jnp.full((_LANES,), d)],
                                   jnp.full((_LANES,), pos16[i]), mask=lane0)
                plsc.store_scatter(startsmut, [jnp.full((_LANES,), l)],
                                   jnp.full((_LANES,), d + 1), mask=lane0)
        return carry

    lax.fori_loop(0, (cnt + _LANES - 1) // _LANES, place_body, 0)

    def seg_start(k):
        graw = _sc1(glist[pl.ds(jnp.minimum(k, 255), _LANES)])
        g = jnp.where(k < ngroups,
                      jnp.minimum(jnp.maximum(graw, 0), 255), 0)
        s = _sc1(starts[pl.ds(g, _LANES)])
        return jnp.where(k < ngroups, s, cnt)

    def group_copy(k, b, phase):
        graw = _sc1(glist[pl.ds(jnp.minimum(k, 255), _LANES)])
        g = jnp.minimum(jnp.maximum(graw, 0), 255)
        off = pl.multiple_of((lo + g) * 128, 128)
        sem = sem_g0 if phase == 0 else sem_g1
        return pltpu.make_async_copy(
            memT_hbm.at[:, pl.ds(off, 128)], gbuf.at[phase, b], sem)

    def fire_wave(w, phase):
        for b in range(_W):
            @pl.when(w * _W + b < ngroups)
            def _():
                group_copy(w * _W + b, b, phase).start()

    def drain_wave(w, phase):
        for b in range(_W):
            @pl.when(w * _W + b < ngroups)
            def _():
                group_copy(w * _W + b, b, phase).wait()

    dummy_cp = pltpu.make_async_copy(idx_hbm.at[pl.ds(0, MEM_DIM)],
                                     dummy_v, sem_out)

    fire_wave(0, 0)

    def drain_n(n):
        def drain_body(i, dcarry):
            dummy_cp.wait()
            return dcarry
        lax.fori_loop(0, n, drain_body, 0)

    def extract_wave(w, phase, rcarry):
        r0 = seg_start(w * _W)
        r1 = seg_start((w + 1) * _W)

        def chunk_body(q, ccarry):
            gq, d0, d1, d2, d3 = ccarry
            r = r0 + q * _LANES
            n_here = jnp.minimum(r1 - r, _LANES)
            mlane = iota16 < n_here
            pos16 = sorted_pos[pl.ds(r, _LANES)]
            possafe = lax.bitwise_and(pos16, BATCH - 1)
            nid16 = plsc.load_gather(idx_all, [possafe])
            local16 = jnp.minimum(jnp.maximum(jnp.where(
                mlane, lax.shift_right_logical(nid16, 7) - lo, 0), 0), 255)
            slot16 = plsc.load_gather(slots, [local16])
            b16 = jnp.minimum(jnp.maximum(
                jnp.where(mlane, slot16 - w * _W, 0), 0), _W - 1)
            dn16 = jnp.where(mlane, lax.bitwise_and(nid16, 127), 0)
            ph16 = jnp.full((_LANES,), phase)
            slot = lax.rem(gq, 4)
            drain_n(d0)
            for f in range(MEM_DIM):
                vals = plsc.load_gather(
                    gbuf, [ph16, b16, jnp.full((_LANES,), f), dn16])
                plsc.store_scatter(rows16.at[slot],
                                   [iota16, jnp.full((_LANES,), f)],
                                   vals, mask=mlane)
            for i in range(_LANES):
                @pl.when(i < n_here)
                def _():
                    pltpu.async_copy(rows16.at[slot, i],
                                     mid_hbm.at[possafe[i]], sem_out)
            return (gq + 1, d1, d2, d3, n_here)

        return lax.fori_loop(0, (r1 - r0 + _LANES - 1) // _LANES,
                             chunk_body, rcarry)

    def pair_body(u, ucarry):
        for ph in range(2):
            w = 2 * u + ph
            fire_wave(w + 1, 1 - ph)
            drain_wave(w, ph)
            ucarry = extract_wave(w, ph, ucarry)
        return ucarry

    _, d0, d1, d2, d3 = lax.fori_loop(0, _MAXWAVES // 2, pair_body,
                                      (0, 0, 0, 0, 0))
    drain_n(d0 + d1 + d2 + d3)


@functools.partial(
    pl.kernel,
    mesh=_mesh,
    out_type=jax.ShapeDtypeStruct((BATCH, MEM_DIM), jnp.float32),
    scratch_types=[
        pltpu.VMEM((_BPW,), jnp.int32),
        pltpu.VMEM((_BPW,), jnp.float32),
        pltpu.VMEM((MEM_DIM,), jnp.float32),
        pltpu.VMEM((_BPW, MEM_DIM), jnp.float32),
        pltpu.SemaphoreType.DMA,
    ],
)
def _default_fixup(mid_hbm, lu_hbm, dflt_hbm, idx_hbm, out_hbm,
                   idx_v, lu_v, dflt_v, rows_v, sem_lu):
    wid = lax.axis_index("s") * _NUM_CORES + lax.axis_index("c")
    base = wid * _BPW

    pltpu.sync_copy(idx_hbm.at[pl.ds(base, _BPW)], idx_v)
    pltpu.sync_copy(dflt_hbm, dflt_v)
    cp_lu = pltpu.async_copy(lu_hbm.at[idx_v], lu_v, sem_lu)
    pltpu.sync_copy(mid_hbm.at[pl.ds(base, _BPW)], rows_v)
    cp_lu.wait()

    dvecs = [dflt_v[pl.ds(j * _LANES, _LANES)] for j in range(_VPR)]

    def chunk_fix(c, carry):
        lu16 = lu_v[pl.ds(c * _LANES, _LANES)]
        for i in range(_LANES):
            @pl.when(lu16[i] == jnp.float32(TIME_INIT))
            def _():
                for j in range(_VPR):
                    rows_v[c * _LANES + i, pl.ds(j * _LANES, _LANES)] = dvecs[j]
        return carry

    lax.fori_loop(0, _BPW // _LANES, chunk_fix, 0)

    pltpu.sync_copy(rows_v, out_hbm.at[pl.ds(base, _BPW)])


def kernel(memory, last_update, default_memory, node_ids):
    idx = node_ids.astype(jnp.int32)
    mid = _route_gather(memory.T, idx)
    return _default_fixup(mid, last_update, default_memory, idx)


# speedup vs baseline: 2.3617x; 1.0605x over previous
import functools

import jax
import jax.numpy as jnp
from jax import lax
from jax.experimental import pallas as pl
from jax.experimental.pallas import tpu as pltpu
from jax.experimental.pallas import tpu_sc as plsc

N_NODES = 1000000
MEM_DIM = 64
BATCH = 16384
TIME_INIT = -1.0

_NUM_CORES = 2
_NUM_SUBCORES = 16
_NW = _NUM_CORES * _NUM_SUBCORES
_BPW = BATCH // _NW
_LANES = 16
_VPR = MEM_DIM // _LANES
_NGRP = (N_NODES + 127) // 128
_MAXOWN = (_NGRP + _NW - 1) // _NW + 1
_W = 3
_MAXWAVES = (_MAXOWN + _W - 1) // _W + 1
_MAXWAVES += _MAXWAVES % 2
_NVREG = BATCH // _LANES
_PAD = 16

_mesh = plsc.VectorSubcoreMesh(core_axis_name="c", subcore_axis_name="s")


def _sc1(v):
    return v[0]


@functools.partial(
    pl.kernel,
    mesh=_mesh,
    compiler_params=pltpu.CompilerParams(needs_layout_passes=False),
    out_type=jax.ShapeDtypeStruct((BATCH, MEM_DIM), jnp.float32),
    scratch_types=[
        pltpu.VMEM((BATCH,), jnp.int32),
        pltpu.VMEM((BATCH + _PAD,), jnp.int32),
        pltpu.VMEM((BATCH + _PAD,), jnp.int32),
        pltpu.VMEM((256 + _PAD,), jnp.int32),
        pltpu.VMEM((256 + _PAD,), jnp.int32),
        pltpu.VMEM((256 + _PAD,), jnp.int32),
        pltpu.VMEM((256 + _PAD,), jnp.int32),
        pltpu.VMEM((256 + _PAD,), jnp.int32),
        pltpu.VMEM((2, _W, MEM_DIM, 128), jnp.float32),
        pltpu.VMEM((4, _LANES, MEM_DIM), jnp.float32),
        pltpu.VMEM((BATCH + _PAD,), jnp.float32),
        pltpu.VMEM((MEM_DIM,), jnp.float32),
        pltpu.VMEM((MEM_DIM,), jnp.int32),
        pltpu.SemaphoreType.DMA,
        pltpu.SemaphoreType.DMA,
        pltpu.SemaphoreType.DMA,
    ],
)
def _route_gather(memT_hbm, lu_hbm, dflt_hbm, idx_hbm, out_hbm,
                  idx_all, poslist, sorted_pos, counts, starts, startsmut,
                  slots, glist, gbuf, rows16, luall, dflt_v, dummy_v,
                  sem_g0, sem_g1, sem_out):
    wid = lax.axis_index("s") * _NUM_CORES + lax.axis_index("c")
    lo = (_NGRP * wid) >> 5
    hi = (_NGRP * (wid + 1)) >> 5

    pltpu.sync_copy(idx_hbm, idx_all)
    pltpu.sync_copy(dflt_hbm, dflt_v)

    iota16 = lax.iota(jnp.int32, _LANES)
    zeros16 = jnp.zeros((_LANES,), jnp.int32)
    ones16 = jnp.full((_LANES,), 1, jnp.int32)
    lane0 = iota16 == 0

    for t in range((256 + _PAD) // _LANES):
        counts[pl.ds(t * _LANES, _LANES)] = zeros16

    def scan_body(c, cnt):
        v16 = idx_all[pl.ds(c * _LANES, _LANES)]
        nc16 = lax.shift_right_logical(v16, 7)
        m = jnp.logical_and(nc16 >= lo, nc16 < hi)
        lsafe = jnp.where(m, nc16 - lo, 0)
        plsc.addupdate_scatter(counts, [lsafe], ones16, mask=m)
        plsc.store_compressed(poslist.at[pl.ds(cnt, _LANES)],
                              c * _LANES + iota16, mask=m)
        return cnt + _sc1(plsc.all_reduce_population_count(m))

    cnt = lax.fori_loop(0, _NVREG, scan_body, 0)

    def prefix_body(t, carry):
        id_carry, g_carry = carry
        cv = counts[pl.ds(t * _LANES, _LANES)]
        inc = plsc.cumsum(cv)
        starts[pl.ds(t * _LANES, _LANES)] = inc - cv + id_carry
        startsmut[pl.ds(t * _LANES, _LANES)] = inc - cv + id_carry
        present = (cv > 0).astype(jnp.int32)
        pinc = plsc.cumsum(present)
        slots[pl.ds(t * _LANES, _LANES)] = pinc - present + g_carry
        plsc.store_compressed(glist.at[pl.ds(g_carry, _LANES)],
                              t * _LANES + iota16, mask=cv > 0)
        return (id_carry + inc[_LANES - 1], g_carry + pinc[_LANES - 1])

    _, ngroups = lax.fori_loop(0, 256 // _LANES, prefix_body, (0, 0))

    def place_body(q, carry):
        n_here = jnp.minimum(cnt - q * _LANES, _LANES)
        pos16 = poslist[pl.ds(q * _LANES, _LANES)]
        nid16 = plsc.load_gather(
            idx_all, [lax.bitwise_and(pos16, BATCH - 1)])
        local16 = jnp.minimum(jnp.maximum(
            lax.shift_right_logical(nid16, 7) - lo, 0), 255)
        for i in range(_LANES):
            @pl.when(i < n_here)
            def _():
                l = local16[i]
                d = lax.bitwise_and(_sc1(startsmut[pl.ds(l, _LANES)]),
                                    BATCH - 1)
                plsc.store_scatter(sorted_pos, [jnp.full((_LANES,), d)],
                                   jnp.full((_LANES,), pos16[i]), mask=lane0)
                plsc.store_scatter(startsmut, [jnp.full((_LANES,), l)],
                                   jnp.full((_LANES,), d + 1), mask=lane0)
        return carry

    lax.fori_loop(0, (cnt + _LANES - 1) // _LANES, place_body, 0)

    nchunks = (cnt + _LANES - 1) // _LANES

    def nid_rewrite(q, carry):
        pos16 = sorted_pos[pl.ds(q * _LANES, _LANES)]
        nid16 = plsc.load_gather(
            idx_all, [lax.bitwise_and(pos16, BATCH - 1)])
        poslist[pl.ds(q * _LANES, _LANES)] = nid16
        return carry

    lax.fori_loop(0, nchunks, nid_rewrite, 0)

    def lu_chunk(q):
        return pltpu.make_async_copy(
            lu_hbm.at[poslist.at[pl.ds(q * _LANES, _LANES)]],
            luall.at[pl.ds(q * _LANES, _LANES)], sem_out)

    def lu_fire(q, carry):
        lu_chunk(q).start()
        return carry

    lax.fori_loop(0, nchunks, lu_fire, 0)

    def lu_drain(q, carry):
        lu_chunk(q).wait()
        return carry

    lax.fori_loop(0, nchunks, lu_drain, 0)

    dvecs = [dflt_v[pl.ds(j * _LANES, _LANES)] for j in range(_VPR)]

    def seg_start(k):
        graw = _sc1(glist[pl.ds(jnp.minimum(k, 255), _LANES)])
        g = jnp.where(k < ngroups,
                      jnp.minimum(jnp.maximum(graw, 0), 255), 0)
        s = _sc1(starts[pl.ds(g, _LANES)])
        return jnp.where(k < ngroups, s, cnt)

    def group_copy(k, b, phase):
        graw = _sc1(glist[pl.ds(jnp.minimum(k, 255), _LANES)])
        g = jnp.minimum(jnp.maximum(graw, 0), 255)
        off = pl.multiple_of((lo + g) * 128, 128)
        sem = sem_g0 if phase == 0 else sem_g1
        return pltpu.make_async_copy(
            memT_hbm.at[:, pl.ds(off, 128)], gbuf.at[phase, b], sem)

    def fire_wave(w, phase):
        for b in range(_W):
            @pl.when(w * _W + b < ngroups)
            def _():
                group_copy(w * _W + b, b, phase).start()

    def drain_wave(w, phase):
        for b in range(_W):
            @pl.when(w * _W + b < ngroups)
            def _():
                group_copy(w * _W + b, b, phase).wait()

    dummy_cp = pltpu.make_async_copy(idx_hbm.at[pl.ds(0, MEM_DIM)],
                                     dummy_v, sem_out)

    fire_wave(0, 0)

    def drain_n(n):
        def drain_body(i, dcarry):
            dummy_cp.wait()
            return dcarry
        lax.fori_loop(0, n, drain_body, 0)

    def extract_wave(w, phase, rcarry):
        r0 = seg_start(w * _W)
        r1 = seg_start((w + 1) * _W)

        def chunk_body(q, ccarry):
            gq, d0, d1, d2, d3 = ccarry
            r = r0 + q * _LANES
            n_here = jnp.minimum(r1 - r, _LANES)
            mlane = iota16 < n_here
            pos16 = sorted_pos[pl.ds(r, _LANES)]
            possafe = lax.bitwise_and(pos16, BATCH - 1)
            nid16 = plsc.load_gather(idx_all, [possafe])
            local16 = jnp.minimum(jnp.maximum(jnp.where(
                mlane, lax.shift_right_logical(nid16, 7) - lo, 0), 0), 255)
            slot16 = plsc.load_gather(slots, [local16])
            b16 = jnp.minimum(jnp.maximum(
                jnp.where(mlane, slot16 - w * _W, 0), 0), _W - 1)
            dn16 = jnp.where(mlane, lax.bitwise_and(nid16, 127), 0)
            ph16 = jnp.full((_LANES,), phase)
            lu16 = luall[pl.ds(r, _LANES)]
            isdflt = jnp.logical_and(lu16 == jnp.float32(TIME_INIT), mlane)
            slot = lax.rem(gq, 4)
            drain_n(d0)
            for f in range(MEM_DIM):
                vals = plsc.load_gather(
                    gbuf, [ph16, b16, jnp.full((_LANES,), f), dn16])
                dsplat = jnp.full((_LANES,), dvecs[f // _LANES][f % _LANES])
                vals = jnp.where(isdflt, dsplat, vals)
                plsc.store_scatter(rows16.at[slot],
                                   [iota16, jnp.full((_LANES,), f)],
                                   vals, mask=mlane)
            for i in range(_LANES):
                @pl.when(i < n_here)
                def _():
                    pltpu.async_copy(rows16.at[slot, i],
                                     out_hbm.at[possafe[i]], sem_out)
            return (gq + 1, d1, d2, d3, n_here)

        return lax.fori_loop(0, (r1 - r0 + _LANES - 1) // _LANES,
                             chunk_body, rcarry)

    def pair_body(u, ucarry):
        for ph in range(2):
            w = 2 * u + ph
            fire_wave(w + 1, 1 - ph)
            drain_wave(w, ph)
            ucarry = extract_wave(w, ph, ucarry)
        return ucarry

    _, d0, d1, d2, d3 = lax.fori_loop(0, _MAXWAVES // 2, pair_body,
                                      (0, 0, 0, 0, 0))
    drain_n(d0 + d1 + d2 + d3)


def kernel(memory, last_update, default_memory, node_ids):
    idx = node_ids.astype(jnp.int32)
    return _route_gather(memory.T, last_update, default_memory, idx)


# speedup vs baseline: 2.3754x; 1.0058x over previous
import functools

import jax
import jax.numpy as jnp
from jax import lax
from jax.experimental import pallas as pl
from jax.experimental.pallas import tpu as pltpu
from jax.experimental.pallas import tpu_sc as plsc

N_NODES = 1000000
MEM_DIM = 64
BATCH = 16384
TIME_INIT = -1.0

_NUM_CORES = 2
_NUM_SUBCORES = 16
_NW = _NUM_CORES * _NUM_SUBCORES
_BPW = BATCH // _NW
_LANES = 16
_VPR = MEM_DIM // _LANES
_NGRP = (N_NODES + 127) // 128
_MAXOWN = (_NGRP + _NW - 1) // _NW + 1
_W = 3
_MAXWAVES = (_MAXOWN + _W - 1) // _W + 1
_MAXWAVES += _MAXWAVES % 2
_NVREG = BATCH // _LANES
_PAD = 16

_mesh = plsc.VectorSubcoreMesh(core_axis_name="c", subcore_axis_name="s")


def _sc1(v):
    return v[0]


@functools.partial(
    pl.kernel,
    mesh=_mesh,
    compiler_params=pltpu.CompilerParams(needs_layout_passes=False),
    out_type=jax.ShapeDtypeStruct((BATCH, MEM_DIM), jnp.float32),
    scratch_types=[
        pltpu.VMEM((BATCH,), jnp.int32),
        pltpu.VMEM((BATCH + _PAD,), jnp.int32),
        pltpu.VMEM((BATCH + _PAD,), jnp.int32),
        pltpu.VMEM((256 + _PAD,), jnp.int32),
        pltpu.VMEM((256 + _PAD,), jnp.int32),
        pltpu.VMEM((256 + _PAD,), jnp.int32),
        pltpu.VMEM((256 + _PAD,), jnp.int32),
        pltpu.VMEM((256 + _PAD,), jnp.int32),
        pltpu.VMEM((2, _W, MEM_DIM, 128), jnp.float32),
        pltpu.VMEM((4, _LANES, MEM_DIM), jnp.float32),
        pltpu.VMEM((BATCH + _PAD,), jnp.float32),
        pltpu.VMEM((MEM_DIM,), jnp.float32),
        pltpu.VMEM((MEM_DIM,), jnp.int32),
        pltpu.SemaphoreType.DMA,
        pltpu.SemaphoreType.DMA,
        pltpu.SemaphoreType.DMA,
    ],
)
def _route_gather(memT_hbm, lu_hbm, dflt_hbm, idx_hbm, out_hbm,
                  idx_all, poslist, sorted_pos, counts, starts, startsmut,
                  slots, glist, gbuf, rows16, luall, dflt_v, dummy_v,
                  sem_g0, sem_g1, sem_out):
    wid = lax.axis_index("s") * _NUM_CORES + lax.axis_index("c")
    lo = (_NGRP * wid) >> 5
    hi = (_NGRP * (wid + 1)) >> 5

    pltpu.sync_copy(idx_hbm, idx_all)
    pltpu.sync_copy(dflt_hbm, dflt_v)

    iota16 = lax.iota(jnp.int32, _LANES)
    zeros16 = jnp.zeros((_LANES,), jnp.int32)
    ones16 = jnp.full((_LANES,), 1, jnp.int32)
    lane0 = iota16 == 0

    for t in range((256 + _PAD) // _LANES):
        counts[pl.ds(t * _LANES, _LANES)] = zeros16

    _UNROLL = 4

    def scan_body(c4, cnt):
        for u in range(_UNROLL):
            c = c4 * _UNROLL + u
            v16 = idx_all[pl.ds(c * _LANES, _LANES)]
            nc16 = lax.shift_right_logical(v16, 7)
            m = jnp.logical_and(nc16 >= lo, nc16 < hi)
            lsafe = jnp.where(m, nc16 - lo, 0)
            plsc.addupdate_scatter(counts, [lsafe], ones16, mask=m)
            plsc.store_compressed(poslist.at[pl.ds(cnt, _LANES)],
                                  c * _LANES + iota16, mask=m)
            cnt = cnt + _sc1(plsc.all_reduce_population_count(m))
        return cnt

    cnt = lax.fori_loop(0, _NVREG // _UNROLL, scan_body, 0)

    def prefix_body(t, carry):
        id_carry, g_carry = carry
        cv = counts[pl.ds(t * _LANES, _LANES)]
        inc = plsc.cumsum(cv)
        starts[pl.ds(t * _LANES, _LANES)] = inc - cv + id_carry
        startsmut[pl.ds(t * _LANES, _LANES)] = inc - cv + id_carry
        present = (cv > 0).astype(jnp.int32)
        pinc = plsc.cumsum(present)
        slots[pl.ds(t * _LANES, _LANES)] = pinc - present + g_carry
        plsc.store_compressed(glist.at[pl.ds(g_carry, _LANES)],
                              t * _LANES + iota16, mask=cv > 0)
        return (id_carry + inc[_LANES - 1], g_carry + pinc[_LANES - 1])

    _, ngroups = lax.fori_loop(0, 256 // _LANES, prefix_body, (0, 0))

    def place_body(q, carry):
        n_here = jnp.minimum(cnt - q * _LANES, _LANES)
        pos16 = poslist[pl.ds(q * _LANES, _LANES)]
        nid16 = plsc.load_gather(
            idx_all, [lax.bitwise_and(pos16, BATCH - 1)])
        local16 = jnp.minimum(jnp.maximum(
            lax.shift_right_logical(nid16, 7) - lo, 0), 255)
        for i in range(_LANES):
            @pl.when(i < n_here)
            def _():
                l = local16[i]
                d = lax.bitwise_and(_sc1(startsmut[pl.ds(l, _LANES)]),
                                    BATCH - 1)
                plsc.store_scatter(sorted_pos, [jnp.full((_LANES,), d)],
                                   jnp.full((_LANES,), pos16[i]), mask=lane0)
                plsc.store_scatter(startsmut, [jnp.full((_LANES,), l)],
                                   jnp.full((_LANES,), d + 1), mask=lane0)
        return carry

    lax.fori_loop(0, (cnt + _LANES - 1) // _LANES, place_body, 0)

    nchunks = (cnt + _LANES - 1) // _LANES

    def nid_rewrite(q, carry):
        pos16 = sorted_pos[pl.ds(q * _LANES, _LANES)]
        nid16 = plsc.load_gather(
            idx_all, [lax.bitwise_and(pos16, BATCH - 1)])
        poslist[pl.ds(q * _LANES, _LANES)] = nid16
        return carry

    lax.fori_loop(0, nchunks, nid_rewrite, 0)

    def lu_chunk(q):
        return pltpu.make_async_copy(
            lu_hbm.at[poslist.at[pl.ds(q * _LANES, _LANES)]],
            luall.at[pl.ds(q * _LANES, _LANES)], sem_out)

    def lu_fire(q, carry):
        lu_chunk(q).start()
        return carry

    lax.fori_loop(0, nchunks, lu_fire, 0)

    dvecs = [dflt_v[pl.ds(j * _LANES, _LANES)] for j in range(_VPR)]

    def seg_start(k):
        graw = _sc1(glist[pl.ds(jnp.minimum(k, 255), _LANES)])
        g = jnp.where(k < ngroups,
                      jnp.minimum(jnp.maximum(graw, 0), 255), 0)
        s = _sc1(starts[pl.ds(g, _LANES)])
        return jnp.where(k < ngroups, s, cnt)

    def group_copy(k, b, phase):
        graw = _sc1(glist[pl.ds(jnp.minimum(k, 255), _LANES)])
        g = jnp.minimum(jnp.maximum(graw, 0), 255)
        off = pl.multiple_of((lo + g) * 128, 128)
        sem = sem_g0 if phase == 0 else sem_g1
        return pltpu.make_async_copy(
            memT_hbm.at[:, pl.ds(off, 128)], gbuf.at[phase, b], sem)

    def fire_wave(w, phase):
        for b in range(_W):
            @pl.when(w * _W + b < ngroups)
            def _():
                group_copy(w * _W + b, b, phase).start()

    def drain_wave(w, phase):
        for b in range(_W):
            @pl.when(w * _W + b < ngroups)
            def _():
                group_copy(w * _W + b, b, phase).wait()

    dummy_cp = pltpu.make_async_copy(idx_hbm.at[pl.ds(0, MEM_DIM)],
                                     dummy_v, sem_out)

    fire_wave(0, 0)

    def lu_drain(q, carry):
        lu_chunk(q).wait()
        return carry

    lax.fori_loop(0, nchunks, lu_drain, 0)

    def drain_n(n):
        def drain_body(i, dcarry):
            dummy_cp.wait()
            return dcarry
        lax.fori_loop(0, n, drain_body, 0)

    def extract_wave(w, phase, rcarry):
        r0 = seg_start(w * _W)
        r1 = seg_start((w + 1) * _W)

        def chunk_body(q, ccarry):
            gq, d0, d1, d2, d3 = ccarry
            r = r0 + q * _LANES
            n_here = jnp.minimum(r1 - r, _LANES)
            mlane = iota16 < n_here
            pos16 = sorted_pos[pl.ds(r, _LANES)]
            possafe = lax.bitwise_and(pos16, BATCH - 1)
            nid16 = plsc.load_gather(idx_all, [possafe])
            local16 = jnp.minimum(jnp.maximum(jnp.where(
                mlane, lax.shift_right_logical(nid16, 7) - lo, 0), 0), 255)
            slot16 = plsc.load_gather(slots, [local16])
            b16 = jnp.minimum(jnp.maximum(
                jnp.where(mlane, slot16 - w * _W, 0), 0), _W - 1)
            dn16 = jnp.where(mlane, lax.bitwise_and(nid16, 127), 0)
            ph16 = jnp.full((_LANES,), phase)
            lu16 = luall[pl.ds(r, _LANES)]
            isdflt = jnp.logical_and(lu16 == jnp.float32(TIME_INIT), mlane)
            slot = lax.rem(gq, 4)
            drain_n(d0)
            for f in range(MEM_DIM):
                vals = plsc.load_gather(
                    gbuf, [ph16, b16, jnp.full((_LANES,), f), dn16])
                dsplat = jnp.full((_LANES,), dvecs[f // _LANES][f % _LANES])
                vals = jnp.where(isdflt, dsplat, vals)
                plsc.store_scatter(rows16.at[slot],
                                   [iota16, jnp.full((_LANES,), f)],
                                   vals, mask=mlane)
            for i in range(_LANES):
                @pl.when(i < n_here)
                def _():
                    pltpu.async_copy(rows16.at[slot, i],
                                     out_hbm.at[possafe[i]], sem_out)
            return (gq + 1, d1, d2, d3, n_here)

        return lax.fori_loop(0, (r1 - r0 + _LANES - 1) // _LANES,
                             chunk_body, rcarry)

    def pair_body(u, ucarry):
        for ph in range(2):
            w = 2 * u + ph
            fire_wave(w + 1, 1 - ph)
            drain_wave(w, ph)
            ucarry = extract_wave(w, ph, ucarry)
        return ucarry

    _, d0, d1, d2, d3 = lax.fori_loop(0, _MAXWAVES // 2, pair_body,
                                      (0, 0, 0, 0, 0))
    drain_n(d0 + d1 + d2 + d3)


def kernel(memory, last_update, default_memory, node_ids):
    idx = node_ids.astype(jnp.int32)
    return _route_gather(memory.T, last_update, default_memory, idx)
